# baseline (speedup 1.0000x reference)
import functools
import math

import numpy as np
import jax
import jax.numpy as jnp
from jax import lax
from jax.experimental import pallas as pl
from jax.experimental.pallas import tpu as pltpu
from jax.experimental.pallas import tpu_sc as plsc

_D = 256
_NH = 8
_NL = 4
_NP = 4
_DH = 32
_SHAPES = np.array([[80, 80], [40, 40], [20, 20], [10, 10]], dtype=np.int64)
_LEN_V = int((_SHAPES[:, 0] * _SHAPES[:, 1]).sum())
_STARTS = np.concatenate([[0], np.cumsum(_SHAPES[:, 0] * _SHAPES[:, 1])[:-1]])

_NW = 32
_TPI = _NL * _NP * 4
_CHUNK_ITEMS = 2
_CHUNK_TERMS = _CHUNK_ITEMS * _TPI

_PERM = np.array(
    [((h * _NL + l) * _NP + p) * 2 + xy
     for h in range(_NH) for xy in range(2)
     for l in range(_NL) for p in range(_NP)],
    dtype=np.int32,
)

_WL = np.repeat(_SHAPES[:, 1].astype(np.float32), _NP).reshape(1, 16)
_HL = np.repeat(_SHAPES[:, 0].astype(np.float32), _NP).reshape(1, 16)
_WLI = np.repeat(_SHAPES[:, 1].astype(np.int32), _NP).reshape(1, 16)
_STI = np.repeat(_STARTS.astype(np.int32), _NP).reshape(1, 16)


def _matmul_bias(x, wt, b, m_blk):
    m, k = x.shape
    n = wt.shape[1]

    def body(x_ref, w_ref, b_ref, o_ref):
        o_ref[...] = (
            jnp.dot(x_ref[...], w_ref[...], preferred_element_type=jnp.float32)
            + b_ref[...]
        )

    return pl.pallas_call(
        body,
        grid=(m // m_blk,),
        in_specs=[
            pl.BlockSpec((m_blk, k), lambda i: (i, 0)),
            pl.BlockSpec((k, n), lambda i: (0, 0)),
            pl.BlockSpec((1, n), lambda i: (0, 0)),
        ],
        out_specs=pl.BlockSpec((m_blk, n), lambda i: (i, 0)),
        out_shape=jax.ShapeDtypeStruct((m, n), jnp.float32),
    )(x, wt, b)


def _matmul_bias_lin128(x, wt, b, m_blk):
    m, k = x.shape
    n = wt.shape[1]
    rows_per_blk = m_blk * n // 128 // 8

    def body(x_ref, w_ref, b_ref, o_ref):
        r = (jnp.dot(x_ref[...], w_ref[...],
                     preferred_element_type=jnp.float32) + b_ref[...])
        o_ref[...] = r.reshape(rows_per_blk, 8, 128)

    return pl.pallas_call(
        body,
        grid=(m // m_blk,),
        in_specs=[
            pl.BlockSpec((m_blk, k), lambda i: (i, 0)),
            pl.BlockSpec((k, n), lambda i: (0, 0)),
            pl.BlockSpec((1, n), lambda i: (0, 0)),
        ],
        out_specs=pl.BlockSpec((rows_per_blk, 8, 128), lambda i: (i, 0, 0)),
        out_shape=jax.ShapeDtypeStruct((m * n // 1024, 8, 128), jnp.float32),
    )(x, wt, b)


def _sampling_params(q2, wofft, boff, wattnt, battn, rbx, rby, boffs):
    nq = q2.shape[0]

    def body(q_ref, wo_ref, bo_ref, wa_ref, ba_ref, rbx_ref, rby_ref,
             bof_ref, wl_ref, hl_ref, wli_ref, sti_ref, idx_ref, wts_ref):
        g = pl.program_id(0)
        wl = wl_ref[...]
        hl = hl_ref[...]
        wli = wli_ref[...]
        sti = sti_ref[...]
        offs = (
            jnp.dot(q_ref[...], wo_ref[0], preferred_element_type=jnp.float32)
            + bo_ref[0]
        )
        attn = (
            jnp.dot(q_ref[...], wa_ref[0], preferred_element_type=jnp.float32)
            + ba_ref[0]
        )
        rbx_v = rbx_ref[...]
        rby_v = rby_ref[...]
        bof_v = bof_ref[...]
        for hh in range(2):
            a = attn[:, hh * 16:(hh + 1) * 16]
            m = jnp.max(a, axis=1, keepdims=True)
            e = jnp.exp(a - m)
            aw = e / jnp.sum(e, axis=1, keepdims=True)
            ox = offs[:, hh * 32:hh * 32 + 16]
            oy = offs[:, hh * 32 + 16:hh * 32 + 32]
            fx = (rbx_v + ox / wl) * wl - 0.5
            fy = (rby_v + oy / hl) * hl - 0.5
            x0 = jnp.floor(fx)
            y0 = jnp.floor(fy)
            wx1 = fx - x0
            wx0 = 1.0 - wx1
            wy1 = fy - y0
            wy0 = 1.0 - wy1
            for c, (cx, cy) in enumerate(((0, 0), (1, 0), (0, 1), (1, 1))):
                xf = x0 + cx
                yf = y0 + cy
                wx = wx1 if cx else wx0
                wy = wy1 if cy else wy0
                valid = ((xf >= 0.0) & (xf <= wl - 1.0)
                         & (yf >= 0.0) & (yf <= hl - 1.0))
                ixc = jnp.clip(xf, 0.0, wl - 1.0).astype(jnp.int32)
                iyc = jnp.clip(yf, 0.0, hl - 1.0).astype(jnp.int32)
                lin = iyc * wli + ixc + sti
                row = bof_v + lin * _NH + (2 * g + hh)
                w = aw * wx * wy * jnp.where(valid, 1.0, 0.0)
                lo = hh * 64 + c * 16
                idx_ref[0, :, lo:lo + 16] = row
                wts_ref[0, :, lo:lo + 16] = w

    return pl.pallas_call(
        body,
        grid=(_NH // 2,),
        in_specs=[
            pl.BlockSpec((nq, _D), lambda g: (0, 0)),
            pl.BlockSpec((1, _D, 64), lambda g: (g, 0, 0)),
            pl.BlockSpec((1, 1, 64), lambda g: (g, 0, 0)),
            pl.BlockSpec((1, _D, 32), lambda g: (g, 0, 0)),
            pl.BlockSpec((1, 1, 32), lambda g: (g, 0, 0)),
            pl.BlockSpec((nq, 16), lambda g: (0, 0)),
            pl.BlockSpec((nq, 16), lambda g: (0, 0)),
            pl.BlockSpec((nq, 1), lambda g: (0, 0)),
            pl.BlockSpec((1, 16), lambda g: (0, 0)),
            pl.BlockSpec((1, 16), lambda g: (0, 0)),
            pl.BlockSpec((1, 16), lambda g: (0, 0)),
            pl.BlockSpec((1, 16), lambda g: (0, 0)),
        ],
        out_specs=(
            pl.BlockSpec((1, nq, 128), lambda g: (g, 0, 0)),
            pl.BlockSpec((1, nq, 128), lambda g: (g, 0, 0)),
        ),
        out_shape=(
            jax.ShapeDtypeStruct((_NH // 2, nq, 128), jnp.int32),
            jax.ShapeDtypeStruct((_NH // 2, nq, 128), jnp.float32),
        ),
    )(q2,
      wofft.reshape(_D, 4, 64).transpose(1, 0, 2),
      boff.reshape(4, 1, 64),
      wattnt.reshape(_D, 4, 32).transpose(1, 0, 2),
      battn.reshape(4, 1, 32),
      rbx, rby, boffs,
      jnp.asarray(_WL), jnp.asarray(_HL), jnp.asarray(_WLI),
      jnp.asarray(_STI))


def _final_proj4(xs, wt, b):
    nq = xs[0].shape[0]
    n = wt.shape[1]

    def body(x0_ref, x1_ref, x2_ref, x3_ref, w_ref, b_ref, o_ref):
        acc = b_ref[...]
        for g, xr in enumerate((x0_ref, x1_ref, x2_ref, x3_ref)):
            acc = acc + jnp.dot(xr[...], w_ref[g * 64:(g + 1) * 64, :],
                                preferred_element_type=jnp.float32)
        o_ref[...] = acc

    return pl.pallas_call(
        body,
        out_shape=jax.ShapeDtypeStruct((nq, n), jnp.float32),
    )(*xs, wt, b)


def _sc_gather_reduce(v_rows, idx3, wts3, items_per_worker):
    chunks = idx3.shape[1]
    mesh = plsc.VectorSubcoreMesh(core_axis_name="c", subcore_axis_name="s")

    @functools.partial(
        pl.kernel,
        out_type=jax.ShapeDtypeStruct((_NW, items_per_worker, _DH),
                                      jnp.float32),
        mesh=mesh,
        scratch_types=[
            pltpu.VMEM((chunks, _CHUNK_TERMS), jnp.int32),
            pltpu.VMEM((chunks, _CHUNK_TERMS), jnp.float32),
            pltpu.VMEM((2, _CHUNK_TERMS, _DH), jnp.float32),
            pltpu.VMEM((items_per_worker, _DH), jnp.float32),
            pltpu.SemaphoreType.DMA,
            pltpu.SemaphoreType.DMA,
        ],
        compiler_params=pltpu.CompilerParams(use_tc_tiling_on_sc=False),
    )
    def k(v_hbm, idx_hbm, wts_hbm, out_hbm, idx_v, wts_v, rows_v, out_v,
          sem0, sem1):
        wid = lax.axis_index("s") * 2 + lax.axis_index("c")
        pltpu.sync_copy(idx_hbm.at[wid], idx_v)
        pltpu.sync_copy(wts_hbm.at[wid], wts_v)

        pltpu.async_copy(v_hbm.at[idx_v.at[0]], rows_v.at[0], sem0)
        pltpu.async_copy(v_hbm.at[idx_v.at[1]], rows_v.at[1], sem1)

        def compute(chunk, buf):
            for it in range(_CHUNK_ITEMS):
                acc0 = jnp.zeros((16,), jnp.float32)
                acc1 = jnp.zeros((16,), jnp.float32)
                for g in range(_TPI // 16):
                    wvec = wts_v[chunk, pl.ds(it * _TPI + g * 16, 16)]
                    for j in range(16):
                        r = it * _TPI + g * 16 + j
                        w = wvec[j]
                        acc0 = acc0 + rows_v[buf, r, pl.ds(0, 16)] * w
                        acc1 = acc1 + rows_v[buf, r, pl.ds(16, 16)] * w
                item = chunk * _CHUNK_ITEMS + it
                out_v[item, pl.ds(0, 16)] = acc0
                out_v[item, pl.ds(16, 16)] = acc1

        def body(t, _):
            c0 = 2 * t
            pltpu.make_async_copy(
                v_hbm.at[idx_v.at[c0]], rows_v.at[0], sem0).wait()
            compute(c0, 0)

            @pl.when(c0 + 2 < chunks)
            def _():
                pltpu.async_copy(
                    v_hbm.at[idx_v.at[c0 + 2]], rows_v.at[0], sem0)

            pltpu.make_async_copy(
                v_hbm.at[idx_v.at[c0 + 1]], rows_v.at[1], sem1).wait()
            compute(c0 + 1, 1)

            @pl.when(c0 + 3 < chunks)
            def _():
                pltpu.async_copy(
                    v_hbm.at[idx_v.at[c0 + 3]], rows_v.at[1], sem1)
            return _

        lax.fori_loop(0, chunks // 2, body, None)
        pltpu.sync_copy(out_v, out_hbm.at[wid])

    return k(v_rows, idx3, wts3)


def kernel(query, refer_bbox, value, value_shapes, W_value, b_value,
           W_off, b_off, W_attn, b_attn, W_out, b_out):
    bs, len_q, d_model = query.shape
    len_v = value.shape[1]
    nq = bs * len_q

    v = _matmul_bias_lin128(value.reshape(bs * len_v, _D), W_value.T,
                            b_value.reshape(1, _D), m_blk=2000)
    v_rows = v.reshape(bs * len_v * _NH, _DH)

    q2 = query.reshape(nq, _D)
    woffp = W_off[_PERM, :]
    boffp = b_off[_PERM]
    rbx = jnp.repeat(refer_bbox[..., 0].reshape(nq, _NL), _NP, axis=1)
    rby = jnp.repeat(refer_bbox[..., 1].reshape(nq, _NL), _NP, axis=1)
    boffs = ((jnp.arange(nq, dtype=jnp.int32) // len_q)
             * (len_v * _NH)).reshape(nq, 1)
    idx, wts = _sampling_params(
        q2, woffp.T, boffp.reshape(1, _D), W_attn.T,
        b_attn.reshape(1, _NH * _NL * _NP), rbx, rby, boffs)

    items = nq * _NH
    ipw = items // _NW
    chunks = ipw // _CHUNK_ITEMS
    idx3 = idx.reshape(_NW, chunks, _CHUNK_TERMS)
    wts3 = wts.reshape(_NW, chunks, _CHUNK_TERMS)
    sampled = _sc_gather_reduce(v_rows, idx3, wts3, ipw)

    s5 = sampled.reshape(_NH // 2, 8, 150, 2, _DH)
    xs = [s5[g].reshape(nq, 2 * _DH) for g in range(_NH // 2)]
    out = _final_proj4(xs, W_out.T, b_out.reshape(1, _D))
    return out.reshape(bs, len_q, d_model)


# speedup vs baseline: 1.2971x; 1.2971x over previous
import functools
import math

import numpy as np
import jax
import jax.numpy as jnp
from jax import lax
from jax.experimental import pallas as pl
from jax.experimental.pallas import tpu as pltpu
from jax.experimental.pallas import tpu_sc as plsc

_D = 256
_NH = 8
_NL = 4
_NP = 4
_DH = 32
_SHAPES = np.array([[80, 80], [40, 40], [20, 20], [10, 10]], dtype=np.int64)
_LEN_V = int((_SHAPES[:, 0] * _SHAPES[:, 1]).sum())
_STARTS = np.concatenate([[0], np.cumsum(_SHAPES[:, 0] * _SHAPES[:, 1])[:-1]])

_NW = 32
_TPI = _NL * _NP * 4
_CHUNK_ITEMS = 2
_CHUNK_TERMS = _CHUNK_ITEMS * _TPI

_PERM = np.array(
    [((h * _NL + l) * _NP + p) * 2 + xy
     for h in range(_NH) for xy in range(2)
     for l in range(_NL) for p in range(_NP)],
    dtype=np.int32,
)

_WL = np.repeat(_SHAPES[:, 1].astype(np.float32), _NP).reshape(1, 16)
_HL = np.repeat(_SHAPES[:, 0].astype(np.float32), _NP).reshape(1, 16)
_WLI = np.repeat(_SHAPES[:, 1].astype(np.int32), _NP).reshape(1, 16)
_STI = np.repeat(_STARTS.astype(np.int32), _NP).reshape(1, 16)


def _matmul_bias(x, wt, b, m_blk):
    m, k = x.shape
    n = wt.shape[1]

    def body(x_ref, w_ref, b_ref, o_ref):
        o_ref[...] = (
            jnp.dot(x_ref[...], w_ref[...], preferred_element_type=jnp.float32)
            + b_ref[...]
        )

    return pl.pallas_call(
        body,
        grid=(m // m_blk,),
        in_specs=[
            pl.BlockSpec((m_blk, k), lambda i: (i, 0)),
            pl.BlockSpec((k, n), lambda i: (0, 0)),
            pl.BlockSpec((1, n), lambda i: (0, 0)),
        ],
        out_specs=pl.BlockSpec((m_blk, n), lambda i: (i, 0)),
        out_shape=jax.ShapeDtypeStruct((m, n), jnp.float32),
    )(x, wt, b)


def _value_proj_lin128(x, wt, b):
    bsz, lv, k = x.shape
    n = wt.shape[1]
    rows_per_blk = lv * n // 1024

    def body(x_ref, w_ref, b_ref, o_ref):
        r = (jnp.dot(x_ref[0], w_ref[...],
                     preferred_element_type=jnp.float32) + b_ref[...])
        o_ref[...] = r.reshape(rows_per_blk, 8, 128)

    return pl.pallas_call(
        body,
        grid=(bsz,),
        in_specs=[
            pl.BlockSpec((1, lv, k), lambda i: (i, 0, 0)),
            pl.BlockSpec((k, n), lambda i: (0, 0)),
            pl.BlockSpec((1, n), lambda i: (0, 0)),
        ],
        out_specs=pl.BlockSpec((rows_per_blk, 8, 128), lambda i: (i, 0, 0)),
        out_shape=jax.ShapeDtypeStruct((bsz * rows_per_blk, 8, 128),
                                       jnp.float32),
    )(x, wt, b)


def _sampling_params(q2, wofft, boff, wattnt, battn, rbx, rby, boffs):
    nq = q2.shape[0]

    def body(q_ref, wo_ref, bo_ref, wa_ref, ba_ref, rbx_ref, rby_ref,
             bof_ref, wl_ref, hl_ref, wli_ref, sti_ref, idx_ref, wts_ref):
        g = pl.program_id(0)
        wl = wl_ref[...]
        hl = hl_ref[...]
        wli = wli_ref[...]
        sti = sti_ref[...]
        offs = (
            jnp.dot(q_ref[...], wo_ref[0], preferred_element_type=jnp.float32)
            + bo_ref[0]
        )
        attn = (
            jnp.dot(q_ref[...], wa_ref[0], preferred_element_type=jnp.float32)
            + ba_ref[0]
        )
        rbx_v = rbx_ref[...]
        rby_v = rby_ref[...]
        bof_v = bof_ref[...]
        for hh in range(2):
            a = attn[:, hh * 16:(hh + 1) * 16]
            m = jnp.max(a, axis=1, keepdims=True)
            e = jnp.exp(a - m)
            aw = e / jnp.sum(e, axis=1, keepdims=True)
            ox = offs[:, hh * 32:hh * 32 + 16]
            oy = offs[:, hh * 32 + 16:hh * 32 + 32]
            fx = (rbx_v + ox / wl) * wl - 0.5
            fy = (rby_v + oy / hl) * hl - 0.5
            x0 = jnp.floor(fx)
            y0 = jnp.floor(fy)
            wx1 = fx - x0
            wx0 = 1.0 - wx1
            wy1 = fy - y0
            wy0 = 1.0 - wy1
            for c, (cx, cy) in enumerate(((0, 0), (1, 0), (0, 1), (1, 1))):
                xf = x0 + cx
                yf = y0 + cy
                wx = wx1 if cx else wx0
                wy = wy1 if cy else wy0
                valid = ((xf >= 0.0) & (xf <= wl - 1.0)
                         & (yf >= 0.0) & (yf <= hl - 1.0))
                ixc = jnp.clip(xf, 0.0, wl - 1.0).astype(jnp.int32)
                iyc = jnp.clip(yf, 0.0, hl - 1.0).astype(jnp.int32)
                lin = iyc * wli + ixc + sti
                row = bof_v + lin * _NH + (2 * g + hh)
                w = aw * wx * wy * jnp.where(valid, 1.0, 0.0)
                lo = hh * 64 + c * 16
                idx_ref[0, :, lo:lo + 16] = row
                wts_ref[0, :, lo:lo + 16] = w

    return pl.pallas_call(
        body,
        grid=(_NH // 2,),
        in_specs=[
            pl.BlockSpec((nq, _D), lambda g: (0, 0)),
            pl.BlockSpec((1, _D, 64), lambda g: (g, 0, 0)),
            pl.BlockSpec((1, 1, 64), lambda g: (g, 0, 0)),
            pl.BlockSpec((1, _D, 32), lambda g: (g, 0, 0)),
            pl.BlockSpec((1, 1, 32), lambda g: (g, 0, 0)),
            pl.BlockSpec((nq, 16), lambda g: (0, 0)),
            pl.BlockSpec((nq, 16), lambda g: (0, 0)),
            pl.BlockSpec((nq, 1), lambda g: (0, 0)),
            pl.BlockSpec((1, 16), lambda g: (0, 0)),
            pl.BlockSpec((1, 16), lambda g: (0, 0)),
            pl.BlockSpec((1, 16), lambda g: (0, 0)),
            pl.BlockSpec((1, 16), lambda g: (0, 0)),
        ],
        out_specs=(
            pl.BlockSpec((1, nq, 128), lambda g: (g, 0, 0)),
            pl.BlockSpec((1, nq, 128), lambda g: (g, 0, 0)),
        ),
        out_shape=(
            jax.ShapeDtypeStruct((_NH // 2, nq, 128), jnp.int32),
            jax.ShapeDtypeStruct((_NH // 2, nq, 128), jnp.float32),
        ),
    )(q2,
      wofft.reshape(_D, 4, 64).transpose(1, 0, 2),
      boff.reshape(4, 1, 64),
      wattnt.reshape(_D, 4, 32).transpose(1, 0, 2),
      battn.reshape(4, 1, 32),
      rbx, rby, boffs,
      jnp.asarray(_WL), jnp.asarray(_HL), jnp.asarray(_WLI),
      jnp.asarray(_STI))


def _final_proj4(xs, wt, b):
    nq = xs[0].shape[0]
    n = wt.shape[1]

    def body(x0_ref, x1_ref, x2_ref, x3_ref, w_ref, b_ref, o_ref):
        acc = b_ref[...]
        for g, xr in enumerate((x0_ref, x1_ref, x2_ref, x3_ref)):
            acc = acc + jnp.dot(xr[...], w_ref[g * 64:(g + 1) * 64, :],
                                preferred_element_type=jnp.float32)
        o_ref[...] = acc

    return pl.pallas_call(
        body,
        out_shape=jax.ShapeDtypeStruct((nq, n), jnp.float32),
    )(*xs, wt, b)


def _sc_gather_reduce(v_rows, idx3, wts3, items_per_worker):
    chunks = idx3.shape[1]
    mesh = plsc.VectorSubcoreMesh(core_axis_name="c", subcore_axis_name="s")

    @functools.partial(
        pl.kernel,
        out_type=jax.ShapeDtypeStruct((_NW, items_per_worker, _DH),
                                      jnp.float32),
        mesh=mesh,
        scratch_types=[
            pltpu.VMEM((chunks, _CHUNK_TERMS), jnp.int32),
            pltpu.VMEM((chunks, _CHUNK_TERMS), jnp.float32),
            pltpu.VMEM((2, _CHUNK_TERMS, _DH), jnp.float32),
            pltpu.VMEM((items_per_worker, _DH), jnp.float32),
            pltpu.SemaphoreType.DMA,
            pltpu.SemaphoreType.DMA,
        ],
        compiler_params=pltpu.CompilerParams(use_tc_tiling_on_sc=False),
    )
    def k(v_hbm, idx_hbm, wts_hbm, out_hbm, idx_v, wts_v, rows_v, out_v,
          sem0, sem1):
        wid = lax.axis_index("s") * 2 + lax.axis_index("c")
        pltpu.sync_copy(idx_hbm.at[wid], idx_v)
        pltpu.sync_copy(wts_hbm.at[wid], wts_v)

        pltpu.async_copy(v_hbm.at[idx_v.at[0]], rows_v.at[0], sem0)
        pltpu.async_copy(v_hbm.at[idx_v.at[1]], rows_v.at[1], sem1)

        def compute(chunk, buf):
            for it in range(_CHUNK_ITEMS):
                acc0 = jnp.zeros((16,), jnp.float32)
                acc1 = jnp.zeros((16,), jnp.float32)
                for g in range(_TPI // 16):
                    wvec = wts_v[chunk, pl.ds(it * _TPI + g * 16, 16)]
                    for j in range(16):
                        r = it * _TPI + g * 16 + j
                        w = wvec[j]
                        acc0 = acc0 + rows_v[buf, r, pl.ds(0, 16)] * w
                        acc1 = acc1 + rows_v[buf, r, pl.ds(16, 16)] * w
                item = chunk * _CHUNK_ITEMS + it
                out_v[item, pl.ds(0, 16)] = acc0
                out_v[item, pl.ds(16, 16)] = acc1

        def body(t, _):
            c0 = 2 * t
            pltpu.make_async_copy(
                v_hbm.at[idx_v.at[c0]], rows_v.at[0], sem0).wait()
            compute(c0, 0)

            @pl.when(c0 + 2 < chunks)
            def _():
                pltpu.async_copy(
                    v_hbm.at[idx_v.at[c0 + 2]], rows_v.at[0], sem0)

            pltpu.make_async_copy(
                v_hbm.at[idx_v.at[c0 + 1]], rows_v.at[1], sem1).wait()
            compute(c0 + 1, 1)

            @pl.when(c0 + 3 < chunks)
            def _():
                pltpu.async_copy(
                    v_hbm.at[idx_v.at[c0 + 3]], rows_v.at[1], sem1)
            return _

        lax.fori_loop(0, chunks // 2, body, None)
        pltpu.sync_copy(out_v, out_hbm.at[wid])

    return k(v_rows, idx3, wts3)


def kernel(query, refer_bbox, value, value_shapes, W_value, b_value,
           W_off, b_off, W_attn, b_attn, W_out, b_out):
    bs, len_q, d_model = query.shape
    len_v = value.shape[1]
    nq = bs * len_q

    v = _value_proj_lin128(value, W_value.T, b_value.reshape(1, _D))
    v_rows = v.reshape(bs * len_v * _NH, _DH)

    q2 = query.reshape(nq, _D)
    woffp = W_off[_PERM, :]
    boffp = b_off[_PERM]
    rbx = jnp.repeat(refer_bbox[..., 0].reshape(nq, _NL), _NP, axis=1)
    rby = jnp.repeat(refer_bbox[..., 1].reshape(nq, _NL), _NP, axis=1)
    boffs = ((jnp.arange(nq, dtype=jnp.int32) // len_q)
             * (len_v * _NH)).reshape(nq, 1)
    idx, wts = _sampling_params(
        q2, woffp.T, boffp.reshape(1, _D), W_attn.T,
        b_attn.reshape(1, _NH * _NL * _NP), rbx, rby, boffs)

    items = nq * _NH
    ipw = items // _NW
    chunks = ipw // _CHUNK_ITEMS
    idx3 = idx.reshape(_NW, chunks, _CHUNK_TERMS)
    wts3 = wts.reshape(_NW, chunks, _CHUNK_TERMS)
    sampled = _sc_gather_reduce(v_rows, idx3, wts3, ipw)

    s5 = sampled.reshape(_NH // 2, 8, 150, 2, _DH)
    xs = [s5[g].reshape(nq, 2 * _DH) for g in range(_NH // 2)]
    out = _final_proj4(xs, W_out.T, b_out.reshape(1, _D))
    return out.reshape(bs, len_q, d_model)


# speedup vs baseline: 1.5017x; 1.1578x over previous
import functools
import math

import numpy as np
import jax
import jax.numpy as jnp
from jax import lax
from jax.experimental import pallas as pl
from jax.experimental.pallas import tpu as pltpu
from jax.experimental.pallas import tpu_sc as plsc

_D = 256
_NH = 8
_NL = 4
_NP = 4
_DH = 32
_SHAPES = np.array([[80, 80], [40, 40], [20, 20], [10, 10]], dtype=np.int64)
_LEN_V = int((_SHAPES[:, 0] * _SHAPES[:, 1]).sum())
_STARTS = np.concatenate([[0], np.cumsum(_SHAPES[:, 0] * _SHAPES[:, 1])[:-1]])

_NW = 32
_TPI = _NL * _NP * 4
_CHUNK_ITEMS = 2
_CHUNK_TERMS = _CHUNK_ITEMS * _TPI

_PERM = np.array(
    [((h * _NL + l) * _NP + p) * 2 + xy
     for h in range(_NH) for xy in range(2)
     for l in range(_NL) for p in range(_NP)],
    dtype=np.int32,
)

_WL = np.repeat(_SHAPES[:, 1].astype(np.float32), _NP).reshape(1, 16)
_HL = np.repeat(_SHAPES[:, 0].astype(np.float32), _NP).reshape(1, 16)
_WLI = np.repeat(_SHAPES[:, 1].astype(np.int32), _NP).reshape(1, 16)
_STI = np.repeat(_STARTS.astype(np.int32), _NP).reshape(1, 16)


def _matmul_bias(x, wt, b, m_blk):
    m, k = x.shape
    n = wt.shape[1]

    def body(x_ref, w_ref, b_ref, o_ref):
        o_ref[...] = (
            jnp.dot(x_ref[...], w_ref[...], preferred_element_type=jnp.float32)
            + b_ref[...]
        )

    return pl.pallas_call(
        body,
        grid=(m // m_blk,),
        in_specs=[
            pl.BlockSpec((m_blk, k), lambda i: (i, 0)),
            pl.BlockSpec((k, n), lambda i: (0, 0)),
            pl.BlockSpec((1, n), lambda i: (0, 0)),
        ],
        out_specs=pl.BlockSpec((m_blk, n), lambda i: (i, 0)),
        out_shape=jax.ShapeDtypeStruct((m, n), jnp.float32),
    )(x, wt, b)


def _value_proj_lin128(x, wt, b, pos_blk):
    lv, bsz, k = x.shape
    n = wt.shape[1]
    rows_per_blk = pos_blk * bsz * n // 1024

    def body(x_ref, w_ref, b_ref, o_ref):
        xb = x_ref[...].reshape(pos_blk * bsz, k)
        r = (jnp.dot(xb, w_ref[...],
                     preferred_element_type=jnp.float32) + b_ref[...])
        o_ref[...] = r.reshape(rows_per_blk, 8, 128)

    return pl.pallas_call(
        body,
        grid=(lv // pos_blk,),
        in_specs=[
            pl.BlockSpec((pos_blk, bsz, k), lambda i: (i, 0, 0)),
            pl.BlockSpec((k, n), lambda i: (0, 0)),
            pl.BlockSpec((1, n), lambda i: (0, 0)),
        ],
        out_specs=pl.BlockSpec((rows_per_blk, 8, 128), lambda i: (i, 0, 0)),
        out_shape=jax.ShapeDtypeStruct((lv * bsz * n // 1024, 8, 128),
                                       jnp.float32),
    )(x, wt, b)


def _sampling_params(q2, wofft, boff, wattnt, battn, rbx, rby, boffs):
    nq = q2.shape[0]

    def body(q_ref, wo_ref, bo_ref, wa_ref, ba_ref, rbx_ref, rby_ref,
             bof_ref, wl_ref, hl_ref, wli_ref, sti_ref, idx_ref, wts_ref):
        g = pl.program_id(0)
        wl = wl_ref[...]
        hl = hl_ref[...]
        wli = wli_ref[...]
        sti = sti_ref[...]
        offs = (
            jnp.dot(q_ref[...], wo_ref[0], preferred_element_type=jnp.float32)
            + bo_ref[0]
        )
        attn = (
            jnp.dot(q_ref[...], wa_ref[0], preferred_element_type=jnp.float32)
            + ba_ref[0]
        )
        rbx_v = rbx_ref[...]
        rby_v = rby_ref[...]
        bof_v = bof_ref[...]
        for hh in range(2):
            a = attn[:, hh * 16:(hh + 1) * 16]
            m = jnp.max(a, axis=1, keepdims=True)
            e = jnp.exp(a - m)
            aw = e / jnp.sum(e, axis=1, keepdims=True)
            ox = offs[:, hh * 32:hh * 32 + 16]
            oy = offs[:, hh * 32 + 16:hh * 32 + 32]
            fx = (rbx_v + ox / wl) * wl - 0.5
            fy = (rby_v + oy / hl) * hl - 0.5
            x0 = jnp.floor(fx)
            y0 = jnp.floor(fy)
            wx1 = fx - x0
            wx0 = 1.0 - wx1
            wy1 = fy - y0
            wy0 = 1.0 - wy1
            for c, (cx, cy) in enumerate(((0, 0), (1, 0), (0, 1), (1, 1))):
                xf = x0 + cx
                yf = y0 + cy
                wx = wx1 if cx else wx0
                wy = wy1 if cy else wy0
                valid = ((xf >= 0.0) & (xf <= wl - 1.0)
                         & (yf >= 0.0) & (yf <= hl - 1.0))
                ixc = jnp.clip(xf, 0.0, wl - 1.0).astype(jnp.int32)
                iyc = jnp.clip(yf, 0.0, hl - 1.0).astype(jnp.int32)
                lin = iyc * wli + ixc + sti
                row = bof_v + lin * (_NH * 4) + (2 * g + hh)
                w = aw * wx * wy * jnp.where(valid, 1.0, 0.0)
                lo = hh * 64 + c * 16
                idx_ref[0, :, lo:lo + 16] = row
                wts_ref[0, :, lo:lo + 16] = w

    return pl.pallas_call(
        body,
        grid=(_NH // 2,),
        in_specs=[
            pl.BlockSpec((nq, _D), lambda g: (0, 0)),
            pl.BlockSpec((1, _D, 64), lambda g: (g, 0, 0)),
            pl.BlockSpec((1, 1, 64), lambda g: (g, 0, 0)),
            pl.BlockSpec((1, _D, 32), lambda g: (g, 0, 0)),
            pl.BlockSpec((1, 1, 32), lambda g: (g, 0, 0)),
            pl.BlockSpec((nq, 16), lambda g: (0, 0)),
            pl.BlockSpec((nq, 16), lambda g: (0, 0)),
            pl.BlockSpec((nq, 1), lambda g: (0, 0)),
            pl.BlockSpec((1, 16), lambda g: (0, 0)),
            pl.BlockSpec((1, 16), lambda g: (0, 0)),
            pl.BlockSpec((1, 16), lambda g: (0, 0)),
            pl.BlockSpec((1, 16), lambda g: (0, 0)),
        ],
        out_specs=(
            pl.BlockSpec((1, nq, 128), lambda g: (g, 0, 0)),
            pl.BlockSpec((1, nq, 128), lambda g: (g, 0, 0)),
        ),
        out_shape=(
            jax.ShapeDtypeStruct((_NH // 2, nq, 128), jnp.int32),
            jax.ShapeDtypeStruct((_NH // 2, nq, 128), jnp.float32),
        ),
    )(q2,
      wofft.reshape(_D, 4, 64).transpose(1, 0, 2),
      boff.reshape(4, 1, 64),
      wattnt.reshape(_D, 4, 32).transpose(1, 0, 2),
      battn.reshape(4, 1, 32),
      rbx, rby, boffs,
      jnp.asarray(_WL), jnp.asarray(_HL), jnp.asarray(_WLI),
      jnp.asarray(_STI))


def _final_proj4(xs, wt, b):
    nq = xs[0].shape[0]
    n = wt.shape[1]

    def body(x0_ref, x1_ref, x2_ref, x3_ref, w_ref, b_ref, o_ref):
        acc = b_ref[...]
        for g, xr in enumerate((x0_ref, x1_ref, x2_ref, x3_ref)):
            acc = acc + jnp.dot(xr[...], w_ref[g * 64:(g + 1) * 64, :],
                                preferred_element_type=jnp.float32)
        o_ref[...] = acc

    return pl.pallas_call(
        body,
        out_shape=jax.ShapeDtypeStruct((nq, n), jnp.float32),
    )(*xs, wt, b)


def _sc_gather_reduce(v_rows, idx3, wts3, items_per_worker):
    chunks = idx3.shape[1]
    mesh = plsc.VectorSubcoreMesh(core_axis_name="c", subcore_axis_name="s")

    @functools.partial(
        pl.kernel,
        out_type=jax.ShapeDtypeStruct((_NW, items_per_worker, _DH),
                                      jnp.float32),
        mesh=mesh,
        scratch_types=[
            pltpu.VMEM((chunks, _CHUNK_TERMS), jnp.int32),
            pltpu.VMEM((chunks, _CHUNK_TERMS), jnp.float32),
            pltpu.VMEM((2, _CHUNK_TERMS, _DH), jnp.float32),
            pltpu.VMEM((items_per_worker, _DH), jnp.float32),
            pltpu.SemaphoreType.DMA,
            pltpu.SemaphoreType.DMA,
        ],
        compiler_params=pltpu.CompilerParams(use_tc_tiling_on_sc=False),
    )
    def k(v_hbm, idx_hbm, wts_hbm, out_hbm, idx_v, wts_v, rows_v, out_v,
          sem0, sem1):
        wid = lax.axis_index("s") * 2 + lax.axis_index("c")
        pltpu.sync_copy(idx_hbm.at[wid], idx_v)
        pltpu.sync_copy(wts_hbm.at[wid], wts_v)

        pltpu.async_copy(v_hbm.at[idx_v.at[0]], rows_v.at[0], sem0)
        pltpu.async_copy(v_hbm.at[idx_v.at[1]], rows_v.at[1], sem1)

        def compute(chunk, buf):
            for it in range(_CHUNK_ITEMS):
                acc0 = jnp.zeros((16,), jnp.float32)
                acc1 = jnp.zeros((16,), jnp.float32)
                for g in range(_TPI // 16):
                    wvec = wts_v[chunk, pl.ds(it * _TPI + g * 16, 16)]
                    for j in range(16):
                        r = it * _TPI + g * 16 + j
                        w = wvec[j]
                        acc0 = acc0 + rows_v[buf, r, pl.ds(0, 16)] * w
                        acc1 = acc1 + rows_v[buf, r, pl.ds(16, 16)] * w
                item = chunk * _CHUNK_ITEMS + it
                out_v[item, pl.ds(0, 16)] = acc0
                out_v[item, pl.ds(16, 16)] = acc1

        def body(t, _):
            c0 = 2 * t
            pltpu.make_async_copy(
                v_hbm.at[idx_v.at[c0]], rows_v.at[0], sem0).wait()
            compute(c0, 0)

            @pl.when(c0 + 2 < chunks)
            def _():
                pltpu.async_copy(
                    v_hbm.at[idx_v.at[c0 + 2]], rows_v.at[0], sem0)

            pltpu.make_async_copy(
                v_hbm.at[idx_v.at[c0 + 1]], rows_v.at[1], sem1).wait()
            compute(c0 + 1, 1)

            @pl.when(c0 + 3 < chunks)
            def _():
                pltpu.async_copy(
                    v_hbm.at[idx_v.at[c0 + 3]], rows_v.at[1], sem1)
            return _

        lax.fori_loop(0, chunks // 2, body, None)
        pltpu.sync_copy(out_v, out_hbm.at[wid])

    return k(v_rows, idx3, wts3)


def kernel(query, refer_bbox, value, value_shapes, W_value, b_value,
           W_off, b_off, W_attn, b_attn, W_out, b_out):
    bs, len_q, d_model = query.shape
    len_v = value.shape[1]
    nq = bs * len_q

    v = _value_proj_lin128(value.transpose(1, 0, 2), W_value.T,
                           b_value.reshape(1, _D), pos_blk=1700)
    v_rows = v.reshape(bs * len_v * _NH, _DH)

    q2 = query.reshape(nq, _D)
    woffp = W_off[_PERM, :]
    boffp = b_off[_PERM]
    rbx = jnp.repeat(refer_bbox[..., 0].reshape(nq, _NL), _NP, axis=1)
    rby = jnp.repeat(refer_bbox[..., 1].reshape(nq, _NL), _NP, axis=1)
    boffs = ((jnp.arange(nq, dtype=jnp.int32) // len_q)
             * _NH).reshape(nq, 1)
    idx, wts = _sampling_params(
        q2, woffp.T, boffp.reshape(1, _D), W_attn.T,
        b_attn.reshape(1, _NH * _NL * _NP), rbx, rby, boffs)

    items = nq * _NH
    ipw = items // _NW
    chunks = ipw // _CHUNK_ITEMS
    idx3 = idx.reshape(_NW, chunks, _CHUNK_TERMS)
    wts3 = wts.reshape(_NW, chunks, _CHUNK_TERMS)
    sampled = _sc_gather_reduce(v_rows, idx3, wts3, ipw)

    s5 = sampled.reshape(_NH // 2, 8, 150, 2, _DH)
    xs = [s5[g].reshape(nq, 2 * _DH) for g in range(_NH // 2)]
    out = _final_proj4(xs, W_out.T, b_out.reshape(1, _D))
    return out.reshape(bs, len_q, d_model)


# speedup vs baseline: 1.5871x; 1.0569x over previous
import functools
import math

import numpy as np
import jax
import jax.numpy as jnp
from jax import lax
from jax.experimental import pallas as pl
from jax.experimental.pallas import tpu as pltpu
from jax.experimental.pallas import tpu_sc as plsc

_D = 256
_NH = 8
_NL = 4
_NP = 4
_DH = 32
_SHAPES = np.array([[80, 80], [40, 40], [20, 20], [10, 10]], dtype=np.int64)
_LEN_V = int((_SHAPES[:, 0] * _SHAPES[:, 1]).sum())
_STARTS = np.concatenate([[0], np.cumsum(_SHAPES[:, 0] * _SHAPES[:, 1])[:-1]])

_NW = 32
_TPI = _NL * _NP * 4
_CHUNK_ITEMS = 2
_CHUNK_TERMS = _CHUNK_ITEMS * _TPI

_PERM = np.array(
    [((h * _NL + l) * _NP + p) * 2 + xy
     for h in range(_NH) for xy in range(2)
     for l in range(_NL) for p in range(_NP)],
    dtype=np.int32,
)

_WL = np.repeat(_SHAPES[:, 1].astype(np.float32), _NP).reshape(1, 16)
_HL = np.repeat(_SHAPES[:, 0].astype(np.float32), _NP).reshape(1, 16)
_WLI = np.repeat(_SHAPES[:, 1].astype(np.int32), _NP).reshape(1, 16)
_STI = np.repeat(_STARTS.astype(np.int32), _NP).reshape(1, 16)


def _matmul_bias(x, wt, b, m_blk):
    m, k = x.shape
    n = wt.shape[1]

    def body(x_ref, w_ref, b_ref, o_ref):
        o_ref[...] = (
            jnp.dot(x_ref[...], w_ref[...], preferred_element_type=jnp.float32)
            + b_ref[...]
        )

    return pl.pallas_call(
        body,
        grid=(m // m_blk,),
        in_specs=[
            pl.BlockSpec((m_blk, k), lambda i: (i, 0)),
            pl.BlockSpec((k, n), lambda i: (0, 0)),
            pl.BlockSpec((1, n), lambda i: (0, 0)),
        ],
        out_specs=pl.BlockSpec((m_blk, n), lambda i: (i, 0)),
        out_shape=jax.ShapeDtypeStruct((m, n), jnp.float32),
    )(x, wt, b)


def _value_proj_lin128(x, wt, b, pos_blk):
    lv, bsz, k = x.shape
    n = wt.shape[1]
    rows_per_blk = pos_blk * bsz * n // 1024

    def body(x_ref, w_ref, b_ref, o_ref):
        xb = x_ref[...].reshape(pos_blk * bsz, k)
        r = (jnp.dot(xb, w_ref[...],
                     preferred_element_type=jnp.float32) + b_ref[...])
        o_ref[...] = r.reshape(rows_per_blk, 8, 128)

    return pl.pallas_call(
        body,
        grid=(lv // pos_blk,),
        in_specs=[
            pl.BlockSpec((pos_blk, bsz, k), lambda i: (i, 0, 0)),
            pl.BlockSpec((k, n), lambda i: (0, 0)),
            pl.BlockSpec((1, n), lambda i: (0, 0)),
        ],
        out_specs=pl.BlockSpec((rows_per_blk, 8, 128), lambda i: (i, 0, 0)),
        out_shape=jax.ShapeDtypeStruct((lv * bsz * n // 1024, 8, 128),
                                       jnp.float32),
    )(x, wt, b)


def _sampling_params(q2, wofft, boff, wattnt, battn, rbx, rby, boffs):
    nq = q2.shape[0]

    def body(q_ref, wo_ref, bo_ref, wa_ref, ba_ref, rbx_ref, rby_ref,
             bof_ref, wl_ref, hl_ref, wli_ref, sti_ref, idx_ref, wts_ref):
        g = pl.program_id(0)
        wl = wl_ref[...]
        hl = hl_ref[...]
        wli = wli_ref[...]
        sti = sti_ref[...]
        offs = (
            jnp.dot(q_ref[...], wo_ref[0], preferred_element_type=jnp.float32)
            + bo_ref[0]
        )
        attn = (
            jnp.dot(q_ref[...], wa_ref[0], preferred_element_type=jnp.float32)
            + ba_ref[0]
        )
        rbx_v = rbx_ref[...]
        rby_v = rby_ref[...]
        bof_v = bof_ref[...]
        for hh in range(2):
            a = attn[:, hh * 16:(hh + 1) * 16]
            m = jnp.max(a, axis=1, keepdims=True)
            e = jnp.exp(a - m)
            aw = e / jnp.sum(e, axis=1, keepdims=True)
            ox = offs[:, hh * 32:hh * 32 + 16]
            oy = offs[:, hh * 32 + 16:hh * 32 + 32]
            fx = (rbx_v + ox / wl) * wl - 0.5
            fy = (rby_v + oy / hl) * hl - 0.5
            x0 = jnp.floor(fx)
            y0 = jnp.floor(fy)
            wx1 = fx - x0
            wx0 = 1.0 - wx1
            wy1 = fy - y0
            wy0 = 1.0 - wy1
            for c, (cx, cy) in enumerate(((0, 0), (1, 0), (0, 1), (1, 1))):
                xf = x0 + cx
                yf = y0 + cy
                wx = wx1 if cx else wx0
                wy = wy1 if cy else wy0
                valid = ((xf >= 0.0) & (xf <= wl - 1.0)
                         & (yf >= 0.0) & (yf <= hl - 1.0))
                ixc = jnp.clip(xf, 0.0, wl - 1.0).astype(jnp.int32)
                iyc = jnp.clip(yf, 0.0, hl - 1.0).astype(jnp.int32)
                lin = iyc * wli + ixc + sti
                row = bof_v + lin * (_NH * 4) + (2 * g + hh)
                w = aw * wx * wy * jnp.where(valid, 1.0, 0.0)
                lo = hh * 64 + c * 16
                idx_ref[0, :, lo:lo + 16] = row
                wts_ref[0, :, lo:lo + 16] = w

    return pl.pallas_call(
        body,
        grid=(_NH // 2,),
        in_specs=[
            pl.BlockSpec((nq, _D), lambda g: (0, 0)),
            pl.BlockSpec((1, _D, 64), lambda g: (g, 0, 0)),
            pl.BlockSpec((1, 1, 64), lambda g: (g, 0, 0)),
            pl.BlockSpec((1, _D, 32), lambda g: (g, 0, 0)),
            pl.BlockSpec((1, 1, 32), lambda g: (g, 0, 0)),
            pl.BlockSpec((nq, 16), lambda g: (0, 0)),
            pl.BlockSpec((nq, 16), lambda g: (0, 0)),
            pl.BlockSpec((nq, 1), lambda g: (0, 0)),
            pl.BlockSpec((1, 16), lambda g: (0, 0)),
            pl.BlockSpec((1, 16), lambda g: (0, 0)),
            pl.BlockSpec((1, 16), lambda g: (0, 0)),
            pl.BlockSpec((1, 16), lambda g: (0, 0)),
        ],
        out_specs=(
            pl.BlockSpec((1, nq, 128), lambda g: (g, 0, 0)),
            pl.BlockSpec((1, nq, 128), lambda g: (g, 0, 0)),
        ),
        out_shape=(
            jax.ShapeDtypeStruct((_NH // 2, nq, 128), jnp.int32),
            jax.ShapeDtypeStruct((_NH // 2, nq, 128), jnp.float32),
        ),
    )(q2,
      wofft.reshape(_D, 4, 64).transpose(1, 0, 2),
      boff.reshape(4, 1, 64),
      wattnt.reshape(_D, 4, 32).transpose(1, 0, 2),
      battn.reshape(4, 1, 32),
      rbx, rby, boffs,
      jnp.asarray(_WL), jnp.asarray(_HL), jnp.asarray(_WLI),
      jnp.asarray(_STI))


def _final_proj4(xs, wt, b):
    nq = xs[0].shape[0]
    n = wt.shape[1]

    def body(x0_ref, x1_ref, x2_ref, x3_ref, w_ref, b_ref, o_ref):
        acc = b_ref[...]
        for g, xr in enumerate((x0_ref, x1_ref, x2_ref, x3_ref)):
            acc = acc + jnp.dot(xr[...], w_ref[g * 64:(g + 1) * 64, :],
                                preferred_element_type=jnp.float32)
        o_ref[...] = acc

    return pl.pallas_call(
        body,
        out_shape=jax.ShapeDtypeStruct((nq, n), jnp.float32),
    )(*xs, wt, b)


def _sc_gather_reduce(v_rows, idx3, wts3, items_per_worker):
    chunks = idx3.shape[1]
    mesh = plsc.VectorSubcoreMesh(core_axis_name="c", subcore_axis_name="s")

    @functools.partial(
        pl.kernel,
        out_type=jax.ShapeDtypeStruct((_NW, items_per_worker, _DH),
                                      jnp.float32),
        mesh=mesh,
        scratch_types=[
            pltpu.VMEM((chunks, _CHUNK_TERMS), jnp.int32),
            pltpu.VMEM((chunks, _CHUNK_TERMS), jnp.float32),
            pltpu.VMEM((3, _CHUNK_TERMS, _DH), jnp.float32),
            pltpu.VMEM((items_per_worker, _DH), jnp.float32),
            pltpu.SemaphoreType.DMA,
            pltpu.SemaphoreType.DMA,
            pltpu.SemaphoreType.DMA,
        ],
        compiler_params=pltpu.CompilerParams(use_tc_tiling_on_sc=False),
    )
    def k(v_hbm, idx_hbm, wts_hbm, out_hbm, idx_v, wts_v, rows_v, out_v,
          sem0, sem1, sem2):
        wid = lax.axis_index("s") * 2 + lax.axis_index("c")
        pltpu.sync_copy(idx_hbm.at[wid], idx_v)
        pltpu.sync_copy(wts_hbm.at[wid], wts_v)
        sems = (sem0, sem1, sem2)

        pltpu.async_copy(v_hbm.at[idx_v.at[0]], rows_v.at[0], sem0)
        pltpu.async_copy(v_hbm.at[idx_v.at[1]], rows_v.at[1], sem1)

        def compute(chunk, buf):
            for it in range(_CHUNK_ITEMS):
                acc0 = jnp.zeros((16,), jnp.float32)
                acc1 = jnp.zeros((16,), jnp.float32)
                for g in range(_TPI // 16):
                    wvec = wts_v[chunk, pl.ds(it * _TPI + g * 16, 16)]
                    for j in range(16):
                        r = it * _TPI + g * 16 + j
                        w = wvec[j]
                        acc0 = acc0 + rows_v[buf, r, pl.ds(0, 16)] * w
                        acc1 = acc1 + rows_v[buf, r, pl.ds(16, 16)] * w
                item = chunk * _CHUNK_ITEMS + it
                out_v[item, pl.ds(0, 16)] = acc0
                out_v[item, pl.ds(16, 16)] = acc1

        def body(t, _):
            for kk in range(3):
                c = 3 * t + kk
                pltpu.make_async_copy(
                    v_hbm.at[idx_v.at[c]], rows_v.at[kk], sems[kk]).wait()

                nb = (kk + 2) % 3

                @pl.when(c + 2 < chunks)
                def _():
                    pltpu.async_copy(
                        v_hbm.at[idx_v.at[c + 2]], rows_v.at[nb], sems[nb])

                compute(c, kk)
            return _

        lax.fori_loop(0, chunks // 3, body, None)
        pltpu.sync_copy(out_v, out_hbm.at[wid])

    return k(v_rows, idx3, wts3)


def kernel(query, refer_bbox, value, value_shapes, W_value, b_value,
           W_off, b_off, W_attn, b_attn, W_out, b_out):
    bs, len_q, d_model = query.shape
    len_v = value.shape[1]
    nq = bs * len_q

    v = _value_proj_lin128(value.transpose(1, 0, 2), W_value.T,
                           b_value.reshape(1, _D), pos_blk=1700)
    v_rows = v.reshape(bs * len_v * _NH, _DH)

    q2 = query.reshape(nq, _D)
    woffp = W_off[_PERM, :]
    boffp = b_off[_PERM]
    rbx = jnp.repeat(refer_bbox[..., 0].reshape(nq, _NL), _NP, axis=1)
    rby = jnp.repeat(refer_bbox[..., 1].reshape(nq, _NL), _NP, axis=1)
    boffs = ((jnp.arange(nq, dtype=jnp.int32) // len_q)
             * _NH).reshape(nq, 1)
    idx, wts = _sampling_params(
        q2, woffp.T, boffp.reshape(1, _D), W_attn.T,
        b_attn.reshape(1, _NH * _NL * _NP), rbx, rby, boffs)

    items = nq * _NH
    ipw = items // _NW
    chunks = ipw // _CHUNK_ITEMS
    idx3 = idx.reshape(_NW, chunks, _CHUNK_TERMS)
    wts3 = wts.reshape(_NW, chunks, _CHUNK_TERMS)
    sampled = _sc_gather_reduce(v_rows, idx3, wts3, ipw)

    s5 = sampled.reshape(_NH // 2, 8, 150, 2, _DH)
    xs = [s5[g].reshape(nq, 2 * _DH) for g in range(_NH // 2)]
    out = _final_proj4(xs, W_out.T, b_out.reshape(1, _D))
    return out.reshape(bs, len_q, d_model)


# speedup vs baseline: 1.6620x; 1.0472x over previous
import functools
import math

import numpy as np
import jax
import jax.numpy as jnp
from jax import lax
from jax.experimental import pallas as pl
from jax.experimental.pallas import tpu as pltpu
from jax.experimental.pallas import tpu_sc as plsc

_D = 256
_NH = 8
_NL = 4
_NP = 4
_DH = 32
_SHAPES = np.array([[80, 80], [40, 40], [20, 20], [10, 10]], dtype=np.int64)
_LEN_V = int((_SHAPES[:, 0] * _SHAPES[:, 1]).sum())
_STARTS = np.concatenate([[0], np.cumsum(_SHAPES[:, 0] * _SHAPES[:, 1])[:-1]])

_NW = 32
_TPI = _NL * _NP * 4
_CHUNK_ITEMS = 2
_CHUNK_TERMS = _CHUNK_ITEMS * _TPI

_PERM = np.array(
    [((h * _NL + l) * _NP + p) * 2 + xy
     for h in range(_NH) for xy in range(2)
     for l in range(_NL) for p in range(_NP)],
    dtype=np.int32,
)

_PERM_EVEN = np.concatenate([np.arange(0, 256, 2), np.arange(1, 256, 2)])
_PERM_OUT = np.concatenate(
    [h * 32 + np.concatenate([np.arange(0, 32, 2), np.arange(1, 32, 2)])
     for h in range(_NH)])

_WL = np.repeat(_SHAPES[:, 1].astype(np.float32), _NP).reshape(1, 16)
_HL = np.repeat(_SHAPES[:, 0].astype(np.float32), _NP).reshape(1, 16)
_WLI = np.repeat(_SHAPES[:, 1].astype(np.int32), _NP).reshape(1, 16)
_STI = np.repeat(_STARTS.astype(np.int32), _NP).reshape(1, 16)


def _matmul_bias(x, wt, b, m_blk):
    m, k = x.shape
    n = wt.shape[1]

    def body(x_ref, w_ref, b_ref, o_ref):
        o_ref[...] = (
            jnp.dot(x_ref[...], w_ref[...], preferred_element_type=jnp.float32)
            + b_ref[...]
        )

    return pl.pallas_call(
        body,
        grid=(m // m_blk,),
        in_specs=[
            pl.BlockSpec((m_blk, k), lambda i: (i, 0)),
            pl.BlockSpec((k, n), lambda i: (0, 0)),
            pl.BlockSpec((1, n), lambda i: (0, 0)),
        ],
        out_specs=pl.BlockSpec((m_blk, n), lambda i: (i, 0)),
        out_shape=jax.ShapeDtypeStruct((m, n), jnp.float32),
    )(x, wt, b)


def _value_proj_lin128(x, wt, b, pos_blk):
    lv, bsz, k = x.shape
    n = wt.shape[1]
    half = n // 2
    rows_per_blk = pos_blk * bsz * half // 1024

    def rne16(f):
        i = jax.lax.bitcast_convert_type(f, jnp.int32)
        return i + jnp.int32(0x7FFF) + ((i >> 16) & jnp.int32(1))

    def body(x_ref, w_ref, b_ref, o_ref):
        xb = x_ref[...].reshape(pos_blk * bsz, k)
        r = (jnp.dot(xb, w_ref[...],
                     preferred_element_type=jnp.float32) + b_ref[...])
        lo = jax.lax.shift_right_logical(rne16(r[:, :half]), 16)
        hi = rne16(r[:, half:]) & jnp.int32(-65536)
        packed = jax.lax.bitcast_convert_type(lo | hi, jnp.float32)
        o_ref[...] = packed.reshape(rows_per_blk, 8, 128)

    return pl.pallas_call(
        body,
        grid=(lv // pos_blk,),
        in_specs=[
            pl.BlockSpec((pos_blk, bsz, k), lambda i: (i, 0, 0)),
            pl.BlockSpec((k, n), lambda i: (0, 0)),
            pl.BlockSpec((1, n), lambda i: (0, 0)),
        ],
        out_specs=pl.BlockSpec((rows_per_blk, 8, 128), lambda i: (i, 0, 0)),
        out_shape=jax.ShapeDtypeStruct((lv * bsz * half // 1024, 8, 128),
                                       jnp.float32),
    )(x, wt, b)


def _sampling_params(q2, wofft, boff, wattnt, battn, rbx, rby, boffs):
    nq = q2.shape[0]

    def body(q_ref, wo_ref, bo_ref, wa_ref, ba_ref, rbx_ref, rby_ref,
             bof_ref, wl_ref, hl_ref, wli_ref, sti_ref, idx_ref, wts_ref):
        g = pl.program_id(0)
        wl = wl_ref[...]
        hl = hl_ref[...]
        wli = wli_ref[...]
        sti = sti_ref[...]
        offs = (
            jnp.dot(q_ref[...], wo_ref[0], preferred_element_type=jnp.float32)
            + bo_ref[0]
        )
        attn = (
            jnp.dot(q_ref[...], wa_ref[0], preferred_element_type=jnp.float32)
            + ba_ref[0]
        )
        rbx_v = rbx_ref[...]
        rby_v = rby_ref[...]
        bof_v = bof_ref[...]
        for hh in range(2):
            a = attn[:, hh * 16:(hh + 1) * 16]
            m = jnp.max(a, axis=1, keepdims=True)
            e = jnp.exp(a - m)
            aw = e / jnp.sum(e, axis=1, keepdims=True)
            ox = offs[:, hh * 32:hh * 32 + 16]
            oy = offs[:, hh * 32 + 16:hh * 32 + 32]
            fx = (rbx_v + ox / wl) * wl - 0.5
            fy = (rby_v + oy / hl) * hl - 0.5
            x0 = jnp.floor(fx)
            y0 = jnp.floor(fy)
            wx1 = fx - x0
            wx0 = 1.0 - wx1
            wy1 = fy - y0
            wy0 = 1.0 - wy1
            for c, (cx, cy) in enumerate(((0, 0), (1, 0), (0, 1), (1, 1))):
                xf = x0 + cx
                yf = y0 + cy
                wx = wx1 if cx else wx0
                wy = wy1 if cy else wy0
                valid = ((xf >= 0.0) & (xf <= wl - 1.0)
                         & (yf >= 0.0) & (yf <= hl - 1.0))
                ixc = jnp.clip(xf, 0.0, wl - 1.0).astype(jnp.int32)
                iyc = jnp.clip(yf, 0.0, hl - 1.0).astype(jnp.int32)
                lin = iyc * wli + ixc + sti
                row = bof_v + lin * (_NH * 4) + (2 * g + hh)
                w = aw * wx * wy * jnp.where(valid, 1.0, 0.0)
                lo = hh * 64 + c * 16
                idx_ref[0, :, lo:lo + 16] = row
                wts_ref[0, :, lo:lo + 16] = w

    return pl.pallas_call(
        body,
        grid=(_NH // 2,),
        in_specs=[
            pl.BlockSpec((nq, _D), lambda g: (0, 0)),
            pl.BlockSpec((1, _D, 64), lambda g: (g, 0, 0)),
            pl.BlockSpec((1, 1, 64), lambda g: (g, 0, 0)),
            pl.BlockSpec((1, _D, 32), lambda g: (g, 0, 0)),
            pl.BlockSpec((1, 1, 32), lambda g: (g, 0, 0)),
            pl.BlockSpec((nq, 16), lambda g: (0, 0)),
            pl.BlockSpec((nq, 16), lambda g: (0, 0)),
            pl.BlockSpec((nq, 1), lambda g: (0, 0)),
            pl.BlockSpec((1, 16), lambda g: (0, 0)),
            pl.BlockSpec((1, 16), lambda g: (0, 0)),
            pl.BlockSpec((1, 16), lambda g: (0, 0)),
            pl.BlockSpec((1, 16), lambda g: (0, 0)),
        ],
        out_specs=(
            pl.BlockSpec((1, nq, 128), lambda g: (g, 0, 0)),
            pl.BlockSpec((1, nq, 128), lambda g: (g, 0, 0)),
        ),
        out_shape=(
            jax.ShapeDtypeStruct((_NH // 2, nq, 128), jnp.int32),
            jax.ShapeDtypeStruct((_NH // 2, nq, 128), jnp.float32),
        ),
    )(q2,
      wofft.reshape(_D, 4, 64).transpose(1, 0, 2),
      boff.reshape(4, 1, 64),
      wattnt.reshape(_D, 4, 32).transpose(1, 0, 2),
      battn.reshape(4, 1, 32),
      rbx, rby, boffs,
      jnp.asarray(_WL), jnp.asarray(_HL), jnp.asarray(_WLI),
      jnp.asarray(_STI))


def _final_proj4(xs, wt, b):
    nq = xs[0].shape[0]
    n = wt.shape[1]

    def body(x0_ref, x1_ref, x2_ref, x3_ref, w_ref, b_ref, o_ref):
        acc = b_ref[...]
        for g, xr in enumerate((x0_ref, x1_ref, x2_ref, x3_ref)):
            acc = acc + jnp.dot(xr[...], w_ref[g * 64:(g + 1) * 64, :],
                                preferred_element_type=jnp.float32)
        o_ref[...] = acc

    return pl.pallas_call(
        body,
        out_shape=jax.ShapeDtypeStruct((nq, n), jnp.float32),
    )(*xs, wt, b)


def _sc_gather_reduce(v_rows, idx3, wts3, items_per_worker):
    chunks = idx3.shape[1]
    mesh = plsc.VectorSubcoreMesh(core_axis_name="c", subcore_axis_name="s")

    @functools.partial(
        pl.kernel,
        out_type=jax.ShapeDtypeStruct((_NW, items_per_worker, _DH),
                                      jnp.float32),
        mesh=mesh,
        scratch_types=[
            pltpu.VMEM((chunks, _CHUNK_TERMS), jnp.int32),
            pltpu.VMEM((chunks, _CHUNK_TERMS), jnp.float32),
            pltpu.VMEM((3, _CHUNK_TERMS, _DH // 2), jnp.float32),
            pltpu.VMEM((items_per_worker, _DH), jnp.float32),
            pltpu.SemaphoreType.DMA,
            pltpu.SemaphoreType.DMA,
            pltpu.SemaphoreType.DMA,
        ],
        compiler_params=pltpu.CompilerParams(use_tc_tiling_on_sc=False),
    )
    def k(v_hbm, idx_hbm, wts_hbm, out_hbm, idx_v, wts_v, rows_v, out_v,
          sem0, sem1, sem2):
        wid = lax.axis_index("s") * 2 + lax.axis_index("c")
        pltpu.sync_copy(idx_hbm.at[wid], idx_v)
        pltpu.sync_copy(wts_hbm.at[wid], wts_v)
        sems = (sem0, sem1, sem2)

        pltpu.async_copy(v_hbm.at[idx_v.at[0]], rows_v.at[0], sem0)
        pltpu.async_copy(v_hbm.at[idx_v.at[1]], rows_v.at[1], sem1)

        def compute(chunk, buf):
            for it in range(_CHUNK_ITEMS):
                acc0 = jnp.zeros((16,), jnp.float32)
                acc1 = jnp.zeros((16,), jnp.float32)
                for g in range(_TPI // 16):
                    wvec = wts_v[chunk, pl.ds(it * _TPI + g * 16, 16)]
                    sixteen = jnp.full((16,), 16, jnp.int32)
                    topmask = jnp.full((16,), -65536, jnp.int32)
                    for j in range(16):
                        r = it * _TPI + g * 16 + j
                        w = wvec[j]
                        wrd = jax.lax.bitcast_convert_type(
                            rows_v[buf, r, :], jnp.int32)
                        lo = jax.lax.bitcast_convert_type(
                            jax.lax.shift_left(wrd, sixteen), jnp.float32)
                        hi = jax.lax.bitcast_convert_type(
                            wrd & topmask, jnp.float32)
                        acc0 = acc0 + lo * w
                        acc1 = acc1 + hi * w
                item = chunk * _CHUNK_ITEMS + it
                out_v[item, pl.ds(0, 16)] = acc0
                out_v[item, pl.ds(16, 16)] = acc1

        def body(t, _):
            for kk in range(3):
                c = 3 * t + kk
                pltpu.make_async_copy(
                    v_hbm.at[idx_v.at[c]], rows_v.at[kk], sems[kk]).wait()

                nb = (kk + 2) % 3

                @pl.when(c + 2 < chunks)
                def _():
                    pltpu.async_copy(
                        v_hbm.at[idx_v.at[c + 2]], rows_v.at[nb], sems[nb])

                compute(c, kk)
            return _

        lax.fori_loop(0, chunks // 3, body, None)
        pltpu.sync_copy(out_v, out_hbm.at[wid])

    return k(v_rows, idx3, wts3)


def kernel(query, refer_bbox, value, value_shapes, W_value, b_value,
           W_off, b_off, W_attn, b_attn, W_out, b_out):
    bs, len_q, d_model = query.shape
    len_v = value.shape[1]
    nq = bs * len_q

    wvp = W_value.T[:, _PERM_EVEN]
    bvp = b_value[_PERM_EVEN].reshape(1, _D)
    v = _value_proj_lin128(value.transpose(1, 0, 2), wvp, bvp, pos_blk=1700)
    v_rows = v.reshape(bs * len_v * _NH, _DH // 2)

    q2 = query.reshape(nq, _D)
    woffp = W_off[_PERM, :]
    boffp = b_off[_PERM]
    rbx = jnp.repeat(refer_bbox[..., 0].reshape(nq, _NL), _NP, axis=1)
    rby = jnp.repeat(refer_bbox[..., 1].reshape(nq, _NL), _NP, axis=1)
    boffs = ((jnp.arange(nq, dtype=jnp.int32) // len_q)
             * _NH).reshape(nq, 1)
    idx, wts = _sampling_params(
        q2, woffp.T, boffp.reshape(1, _D), W_attn.T,
        b_attn.reshape(1, _NH * _NL * _NP), rbx, rby, boffs)

    items = nq * _NH
    ipw = items // _NW
    chunks = ipw // _CHUNK_ITEMS
    idx3 = idx.reshape(_NW, chunks, _CHUNK_TERMS)
    wts3 = wts.reshape(_NW, chunks, _CHUNK_TERMS)
    sampled = _sc_gather_reduce(v_rows, idx3, wts3, ipw)

    s5 = sampled.reshape(_NH // 2, 8, 150, 2, _DH)
    xs = [s5[g].reshape(nq, 2 * _DH) for g in range(_NH // 2)]
    out = _final_proj4(xs, W_out.T[_PERM_OUT], b_out.reshape(1, _D))
    return out.reshape(bs, len_q, d_model)
